# baseline (speedup 1.0000x reference)
import functools

import jax
import jax.numpy as jnp
from jax import lax
from jax.experimental import pallas as pl
from jax.experimental.pallas import tpu as pltpu
from jax.experimental.pallas import tpu_sc as plsc

_VOCAB = 100000
_DIM = 16
_BATCH = 1024
_CTX = 50

_NC, _NS = 2, 16
_NW = _NC * _NS
_CHUNK = 128


def _make_emb_body(bpw, nchunk):
    gpw = bpw * _CTX

    def _emb_body(idx_hbm, table_hbm, out_hbm, idx_v, rows_v, acc_v, sem):
        wid = lax.axis_index("s") * _NC + lax.axis_index("c")
        pltpu.sync_copy(idx_hbm.at[wid], idx_v)
        copies = []
        for c in range(nchunk):
            copies.append(
                pltpu.make_async_copy(
                    table_hbm.at[idx_v.at[c]],
                    rows_v.at[pl.ds(c * _CHUNK, _CHUNK)],
                    sem,
                )
            )
        for cp in copies:
            cp.start()
        for cp in copies:
            cp.wait()

        def body(r, carry):
            base = r * _CTX
            acc = rows_v[base]
            for j in range(1, _CTX):
                acc = acc + rows_v[base + j]
            acc_v[r] = acc
            return carry

        lax.fori_loop(0, bpw, body, 0)
        pltpu.sync_copy(acc_v, out_hbm.at[pl.ds(wid * bpw, bpw)])

    return _emb_body


def _embed(words, table):
    nrows = words.shape[0]
    bpw = nrows // _NW
    gpw = bpw * _CTX
    nchunk = (gpw + _CHUNK - 1) // _CHUNK
    gpad = nchunk * _CHUNK
    idx = words.reshape(_NW, gpw).astype(jnp.int32)
    idx = jnp.pad(idx, ((0, 0), (0, gpad - gpw))).reshape(_NW, nchunk, _CHUNK)
    mesh = plsc.VectorSubcoreMesh(core_axis_name="c", subcore_axis_name="s")
    f = functools.partial(
        pl.kernel,
        mesh=mesh,
        out_type=jax.ShapeDtypeStruct((nrows, _DIM), jnp.float32),
        scratch_types=[
            pltpu.VMEM((nchunk, _CHUNK), jnp.int32),
            pltpu.VMEM((gpad, _DIM), jnp.float32),
            pltpu.VMEM((bpw, _DIM), jnp.float32),
            pltpu.SemaphoreType.DMA,
        ],
        compiler_params=pltpu.CompilerParams(use_tc_tiling_on_sc=False),
    )(_make_emb_body(bpw, nchunk))
    return f(idx, table)


_RB = 64
_NRSTEP = _BATCH // _RB
_NBUF = 2
_VA = (_VOCAB // 128) * 128
_TW = _VOCAB - _VA


def _proj_body(emb_ref, w_ref, b_ref, out_hbm, obuf, tbuf, sems, tsems):
    i = pl.program_id(0)
    slot = lax.rem(i, _NBUF)

    def _waits(j, s):
        pltpu.make_async_copy(
            obuf.at[s],
            out_hbm.at[pl.ds(j * _RB, _RB), pl.ds(0, _VA)],
            sems.at[s],
        ).wait()
        pltpu.make_async_copy(
            tbuf.at[s],
            out_hbm.at[pl.ds(j * _RB, _RB), pl.ds(_VA, _TW)],
            tsems.at[s],
        ).wait()

    @pl.when(i >= _NBUF)
    def _():
        _waits(i - _NBUF, slot)

    val = (
        lax.dot_general(
            emb_ref[...],
            w_ref[...],
            dimension_numbers=(((1,), (0,)), ((), ())),
            preferred_element_type=jnp.float32,
        )
        + b_ref[...]
    )
    obuf[slot] = val[:, :_VA]
    tbuf[slot] = val[:, _VA:]
    pltpu.async_copy(
        obuf.at[slot],
        out_hbm.at[pl.ds(i * _RB, _RB), pl.ds(0, _VA)],
        sems.at[slot],
    )
    pltpu.async_copy(
        tbuf.at[slot],
        out_hbm.at[pl.ds(i * _RB, _RB), pl.ds(_VA, _TW)],
        tsems.at[slot],
    )

    @pl.when(i == _NRSTEP - 1)
    def _():
        for j in range(_NRSTEP - _NBUF, _NRSTEP):
            _waits(j, j % _NBUF)


def _project(emb, wt, b2):
    return pl.pallas_call(
        _proj_body,
        grid=(_NRSTEP,),
        in_specs=[
            pl.BlockSpec((_RB, _DIM), lambda i: (i, 0)),
            pl.BlockSpec((_DIM, _VOCAB), lambda i: (0, 0)),
            pl.BlockSpec((1, _VOCAB), lambda i: (0, 0)),
        ],
        out_specs=pl.BlockSpec(memory_space=pltpu.MemorySpace.HBM),
        out_shape=jax.ShapeDtypeStruct((_BATCH, _VOCAB), jnp.float32),
        scratch_shapes=[
            pltpu.VMEM((_NBUF, _RB, _VA), jnp.float32),
            pltpu.VMEM((_NBUF, _RB, _TW), jnp.float32),
            pltpu.SemaphoreType.DMA((_NBUF,)),
            pltpu.SemaphoreType.DMA((_NBUF,)),
        ],
        compiler_params=pltpu.CompilerParams(
            dimension_semantics=("arbitrary",),
            disable_bounds_checks=True,
        ),
    )(emb, wt, b2)


def kernel(words, table, W, b):
    wt = W.T
    b2 = b.reshape(1, _VOCAB)
    emb = _embed(words, table)
    return _project(emb, wt, b2)


# speedup vs baseline: 1.0036x; 1.0036x over previous
import functools

import jax
import jax.numpy as jnp
from jax import lax
from jax.experimental import pallas as pl
from jax.experimental.pallas import tpu as pltpu
from jax.experimental.pallas import tpu_sc as plsc

_VOCAB = 100000
_DIM = 16
_BATCH = 1024
_CTX = 50

_NC, _NS = 2, 16
_NW = _NC * _NS
_CHUNK = 128


def _make_emb_body(bpw, nchunk):
    gpw = bpw * _CTX

    def _emb_body(idx_hbm, table_hbm, out_hbm, idx_v, rows_v, acc_v, sem):
        wid = lax.axis_index("s") * _NC + lax.axis_index("c")
        pltpu.sync_copy(idx_hbm.at[wid], idx_v)
        copies = []
        for c in range(nchunk):
            copies.append(
                pltpu.make_async_copy(
                    table_hbm.at[idx_v.at[c]],
                    rows_v.at[pl.ds(c * _CHUNK, _CHUNK)],
                    sem,
                )
            )
        for cp in copies:
            cp.start()
        for cp in copies:
            cp.wait()

        def body(r, carry):
            base = r * _CTX
            acc = rows_v[base]
            for j in range(1, _CTX):
                acc = acc + rows_v[base + j]
            acc_v[r] = acc
            return carry

        lax.fori_loop(0, bpw, body, 0)
        pltpu.sync_copy(acc_v, out_hbm.at[pl.ds(wid * bpw, bpw)])

    return _emb_body


def _embed(words, table):
    nrows = words.shape[0]
    bpw = nrows // _NW
    gpw = bpw * _CTX
    nchunk = (gpw + _CHUNK - 1) // _CHUNK
    gpad = nchunk * _CHUNK
    idx = words.reshape(_NW, gpw).astype(jnp.int32)
    idx = jnp.pad(idx, ((0, 0), (0, gpad - gpw))).reshape(_NW, nchunk, _CHUNK)
    mesh = plsc.VectorSubcoreMesh(core_axis_name="c", subcore_axis_name="s")
    f = functools.partial(
        pl.kernel,
        mesh=mesh,
        out_type=jax.ShapeDtypeStruct((nrows, _DIM), jnp.float32),
        scratch_types=[
            pltpu.VMEM((nchunk, _CHUNK), jnp.int32),
            pltpu.VMEM((gpad, _DIM), jnp.float32),
            pltpu.VMEM((bpw, _DIM), jnp.float32),
            pltpu.SemaphoreType.DMA,
        ],
        compiler_params=pltpu.CompilerParams(use_tc_tiling_on_sc=False),
    )(_make_emb_body(bpw, nchunk))
    return f(idx, table)


_RB = 64
_NRSTEP = _BATCH // _RB
_NBUF = 2
_VA = (_VOCAB // 128) * 128
_TW = _VOCAB - _VA


def _proj_body(emb_ref, w_ref, b_ref, out_hbm, obuf, tbuf, sems, tsems):
    i = pl.program_id(0)
    slot = lax.rem(i, _NBUF)

    def _waits(j, s):
        for k in range(_RB // 8):
            pltpu.make_async_copy(
                obuf.at[s, pl.ds(k * 8, 8)],
                out_hbm.at[pl.ds(j * _RB + k * 8, 8), pl.ds(0, _VA)],
                sems.at[s],
            ).wait()
        pltpu.make_async_copy(
            tbuf.at[s],
            out_hbm.at[pl.ds(j * _RB, _RB), pl.ds(_VA, _TW)],
            tsems.at[s],
        ).wait()

    @pl.when(i >= _NBUF)
    def _():
        _waits(i - _NBUF, slot)

    val = (
        lax.dot_general(
            emb_ref[...],
            w_ref[...],
            dimension_numbers=(((1,), (0,)), ((), ())),
            preferred_element_type=jnp.float32,
        )
        + b_ref[...]
    )
    obuf[slot] = val[:, :_VA]
    tbuf[slot] = val[:, _VA:]
    for k in range(_RB // 8):
        pltpu.async_copy(
            obuf.at[slot, pl.ds(k * 8, 8)],
            out_hbm.at[pl.ds(i * _RB + k * 8, 8), pl.ds(0, _VA)],
            sems.at[slot],
        )
    pltpu.async_copy(
        tbuf.at[slot],
        out_hbm.at[pl.ds(i * _RB, _RB), pl.ds(_VA, _TW)],
        tsems.at[slot],
    )

    @pl.when(i == _NRSTEP - 1)
    def _():
        for j in range(_NRSTEP - _NBUF, _NRSTEP):
            _waits(j, j % _NBUF)


def _project(emb, wt, b2):
    return pl.pallas_call(
        _proj_body,
        grid=(_NRSTEP,),
        in_specs=[
            pl.BlockSpec((_RB, _DIM), lambda i: (i, 0)),
            pl.BlockSpec((_DIM, _VOCAB), lambda i: (0, 0)),
            pl.BlockSpec((1, _VOCAB), lambda i: (0, 0)),
        ],
        out_specs=pl.BlockSpec(memory_space=pltpu.MemorySpace.HBM),
        out_shape=jax.ShapeDtypeStruct((_BATCH, _VOCAB), jnp.float32),
        scratch_shapes=[
            pltpu.VMEM((_NBUF, _RB, _VA), jnp.float32),
            pltpu.VMEM((_NBUF, _RB, _TW), jnp.float32),
            pltpu.SemaphoreType.DMA((_NBUF,)),
            pltpu.SemaphoreType.DMA((_NBUF,)),
        ],
        compiler_params=pltpu.CompilerParams(
            dimension_semantics=("arbitrary",),
            disable_bounds_checks=True,
        ),
    )(emb, wt, b2)


def kernel(words, table, W, b):
    wt = W.T
    b2 = b.reshape(1, _VOCAB)
    emb = _embed(words, table)
    return _project(emb, wt, b2)
